# baseline (device time: 15259 ns/iter reference)
import functools

import jax
import jax.numpy as jnp
from jax import lax
from jax.experimental import pallas as pl
from jax.experimental.pallas import tpu as pltpu

N_DEV = 4
N_TOK = 512
D_IN = 256
D_OUT = 512
N_EXP = 8
EXP_PER_DEV = 2
CHUNK = N_TOK // N_DEV


def kernel(x, router_W, route_idx, expert_W):
    def body(x_ref, rw_ref, idx_ref, ew_ref, out_ref,
             xs_ref, ew_bf, send_buf, recv_buf, send_sems, recv_sems):
        my = lax.axis_index("i")

        bar = pltpu.get_barrier_semaphore()
        for k in range(1, N_DEV):
            pl.semaphore_signal(bar, inc=1,
                                device_id=(lax.rem(my + k, N_DEV),),
                                device_id_type=pl.DeviceIdType.MESH)
        pl.semaphore_wait(bar, N_DEV - 1)

        xf = x_ref[:, :]
        scores = jnp.dot(xf, rw_ref[:, :], preferred_element_type=jnp.float32)
        s_max = jnp.max(scores, axis=-1, keepdims=True)
        p = jnp.exp(scores - s_max)
        probs = p / jnp.sum(p, axis=-1, keepdims=True)

        idx0 = idx_ref[:, 0:1]
        idx1 = idx_ref[:, 1:2]
        eids = lax.broadcasted_iota(jnp.int32, (N_TOK, N_EXP), 1)
        g0 = jnp.sum(jnp.where(eids == idx0, probs, 0.0), axis=1, keepdims=True)
        g1 = jnp.sum(jnp.where(eids == idx1, probs, 0.0), axis=1, keepdims=True)
        gs = g0 + g1

        for j in range(EXP_PER_DEV):
            e = my * EXP_PER_DEV + j
            pe = jnp.sum(jnp.where(eids == e, probs, 0.0), axis=1, keepdims=True)
            hit = jnp.logical_or(idx0 == e, idx1 == e).astype(jnp.float32)
            w = pe / gs * hit
            xs_ref[j] = (w * xf).astype(jnp.bfloat16)
            ew_bf[j] = ew_ref[j].astype(jnp.bfloat16)

        rdmas = []
        for k in range(1, N_DEV):
            t = lax.rem(my + k, N_DEV)
            x0 = xs_ref[0, pl.ds(t * CHUNK, CHUNK), :]
            x1 = xs_ref[1, pl.ds(t * CHUNK, CHUNK), :]
            yc = (jnp.dot(x0, ew_bf[0], preferred_element_type=jnp.float32)
                  + jnp.dot(x1, ew_bf[1], preferred_element_type=jnp.float32))
            send_buf[k - 1] = yc.astype(jnp.bfloat16)
            rdma = pltpu.make_async_remote_copy(
                src_ref=send_buf.at[k - 1],
                dst_ref=recv_buf.at[N_DEV - 1 - k],
                send_sem=send_sems.at[k - 1],
                recv_sem=recv_sems.at[N_DEV - 1 - k],
                device_id=(t,),
                device_id_type=pl.DeviceIdType.MESH,
            )
            rdma.start()
            rdmas.append(rdma)

        x0 = xs_ref[0, pl.ds(my * CHUNK, CHUNK), :]
        x1 = xs_ref[1, pl.ds(my * CHUNK, CHUNK), :]
        total = (jnp.dot(x0, ew_bf[0], preferred_element_type=jnp.float32)
                 + jnp.dot(x1, ew_bf[1], preferred_element_type=jnp.float32))

        for rdma in rdmas:
            rdma.wait_recv()
        for j in range(N_DEV - 1):
            total = total + recv_buf[j].astype(jnp.float32)
        out_ref[:, :] = total

        for rdma in rdmas:
            rdma.wait_send()

        @functools.partial(pl.run_scoped, sem=pltpu.SemaphoreType.REGULAR)
        def _(sem):
            for k in range(1, N_DEV):
                pl.semaphore_signal(sem, inc=1,
                                    device_id=(lax.rem(my + k, N_DEV),),
                                    device_id_type=pl.DeviceIdType.MESH)
            pl.semaphore_wait(sem, N_DEV - 1)

    return pl.pallas_call(
        body,
        out_shape=jax.ShapeDtypeStruct((CHUNK, D_OUT), jnp.float32),
        in_specs=[pl.BlockSpec(memory_space=pltpu.VMEM)] * 4,
        out_specs=pl.BlockSpec(memory_space=pltpu.VMEM),
        scratch_shapes=[
            pltpu.VMEM((EXP_PER_DEV, N_TOK, D_IN), jnp.bfloat16),
            pltpu.VMEM((EXP_PER_DEV, D_IN, D_OUT), jnp.bfloat16),
            pltpu.VMEM((N_DEV - 1, CHUNK, D_OUT), jnp.bfloat16),
            pltpu.VMEM((N_DEV - 1, CHUNK, D_OUT), jnp.bfloat16),
            pltpu.SemaphoreType.DMA((N_DEV - 1,)),
            pltpu.SemaphoreType.DMA((N_DEV - 1,)),
        ],
        compiler_params=pltpu.CompilerParams(collective_id=0),
    )(x, router_W, route_idx, expert_W)


# device time: 13564 ns/iter; 1.1250x vs baseline; 1.1250x over previous
import jax
import jax.numpy as jnp
from jax import lax
from jax.experimental import pallas as pl
from jax.experimental.pallas import tpu as pltpu

N_DEV = 4
N_TOK = 512
D_IN = 256
D_OUT = 512
N_EXP = 8
EXP_PER_DEV = 2
CHUNK = N_TOK // N_DEV


def kernel(x, router_W, route_idx, expert_W):
    xb = x.astype(jnp.bfloat16)
    rwb = router_W.astype(jnp.bfloat16)
    ewb = expert_W.astype(jnp.bfloat16).reshape(EXP_PER_DEV * D_IN, D_OUT)

    def body(x_ref, rw_ref, idx_ref, ew_ref, out_ref,
             xs_ref, send_buf, recv_buf, send_sems, recv_sems):
        my = lax.axis_index("i")

        xf = x_ref[:, :]
        scores = jnp.dot(xf, rw_ref[:, :], preferred_element_type=jnp.float32)
        s_max = jnp.max(scores, axis=-1, keepdims=True)
        p = jnp.exp(scores - s_max)

        idx0 = idx_ref[:, 0:1]
        idx1 = idx_ref[:, 1:2]
        eids = lax.broadcasted_iota(jnp.int32, (N_TOK, N_EXP), 1)
        g0 = jnp.sum(jnp.where(eids == idx0, p, 0.0), axis=1, keepdims=True)
        g1 = jnp.sum(jnp.where(eids == idx1, p, 0.0), axis=1, keepdims=True)
        gs = g0 + g1

        for j in range(EXP_PER_DEV):
            e = my * EXP_PER_DEV + j
            pe = jnp.sum(jnp.where(eids == e, p, 0.0), axis=1, keepdims=True)
            hit = jnp.logical_or(idx0 == e, idx1 == e).astype(jnp.float32)
            w = pe / gs * hit
            xs_ref[:, j * D_IN:(j + 1) * D_IN] = xf * w.astype(jnp.bfloat16)

        for k in range(1, N_DEV):
            t = lax.rem(my + k, N_DEV)
            xc = xs_ref[pl.ds(t * CHUNK, CHUNK), :]
            yc = jnp.dot(xc, ew_ref[:, :], preferred_element_type=jnp.float32)
            send_buf[k - 1] = yc.astype(jnp.bfloat16)

        bar = pltpu.get_barrier_semaphore()
        for k in range(1, N_DEV):
            pl.semaphore_signal(bar, inc=1,
                                device_id=(lax.rem(my + k, N_DEV),),
                                device_id_type=pl.DeviceIdType.MESH)
        pl.semaphore_wait(bar, N_DEV - 1)

        rdmas = []
        for k in range(1, N_DEV):
            t = lax.rem(my + k, N_DEV)
            rdma = pltpu.make_async_remote_copy(
                src_ref=send_buf.at[k - 1],
                dst_ref=recv_buf.at[N_DEV - 1 - k],
                send_sem=send_sems.at[k - 1],
                recv_sem=recv_sems.at[N_DEV - 1 - k],
                device_id=(t,),
                device_id_type=pl.DeviceIdType.MESH,
            )
            rdma.start()
            rdmas.append(rdma)

        xc = xs_ref[pl.ds(my * CHUNK, CHUNK), :]
        total = jnp.dot(xc, ew_ref[:, :], preferred_element_type=jnp.float32)

        for rdma in rdmas:
            rdma.wait_recv()
        for j in range(N_DEV - 1):
            total = total + recv_buf[j].astype(jnp.float32)
        out_ref[:, :] = total

        for rdma in rdmas:
            rdma.wait_send()

    return pl.pallas_call(
        body,
        out_shape=jax.ShapeDtypeStruct((CHUNK, D_OUT), jnp.float32),
        in_specs=[pl.BlockSpec(memory_space=pltpu.VMEM)] * 4,
        out_specs=pl.BlockSpec(memory_space=pltpu.VMEM),
        scratch_shapes=[
            pltpu.VMEM((N_TOK, EXP_PER_DEV * D_IN), jnp.bfloat16),
            pltpu.VMEM((N_DEV - 1, CHUNK, D_OUT), jnp.bfloat16),
            pltpu.VMEM((N_DEV - 1, CHUNK, D_OUT), jnp.bfloat16),
            pltpu.SemaphoreType.DMA((N_DEV - 1,)),
            pltpu.SemaphoreType.DMA((N_DEV - 1,)),
        ],
        compiler_params=pltpu.CompilerParams(collective_id=0),
    )(xb, rwb, route_idx, ewb)
